# baseline (speedup 1.0000x reference)
import functools

import jax
import jax.numpy as jnp
from jax import lax
from jax.experimental import pallas as pl
from jax.experimental import layout as jlayout
from jax.experimental.pallas import tpu as pltpu
from jax.experimental.pallas import tpu_sc as plsc

VOCAB_SIZE = 1000000
EMBED_DIM = 64
BATCH = 16384
HIST_LEN = 50

_INFO = plsc.get_sparse_core_info()
_NC, _NS = _INFO.num_cores, _INFO.num_subcores
_NW = _NC * _NS
_RPW = BATCH // _NW
_BT = BATCH // 128
_BTPW = _BT // _NW
_CT = 256
_NCHUNK = HIST_LEN * 2
_RING = 2


def _make_sc_gather():
  mesh = plsc.VectorSubcoreMesh(core_axis_name="c", subcore_axis_name="s")

  @functools.partial(
      pl.kernel,
      mesh=mesh,
      compiler_params=pltpu.CompilerParams(use_tc_tiling_on_sc=False,
                                           needs_layout_passes=False),
      out_type=jax.ShapeDtypeStruct((HIST_LEN, 8, 128, 8, 128), jnp.float32),
      scratch_types=[
          pltpu.VMEM((HIST_LEN, _RPW), jnp.int32),
          pltpu.VMEM((_RING, _CT, EMBED_DIM), jnp.float32),
          pltpu.VMEM((_RING, 8, 2, 8, 128), jnp.float32),
          [pltpu.SemaphoreType.DMA] * _RING,
          [pltpu.SemaphoreType.DMA] * _RING,
      ],
  )
  def k(table_hbm, idxt_hbm, out_hbm, idx_v, rows_v, t_v, gsem, ssem):
    wid = lax.axis_index("s") * _NC + lax.axis_index("c")
    b0 = wid * _RPW
    bt0 = wid * _BTPW
    pltpu.sync_copy(idxt_hbm.at[:, pl.ds(b0, _RPW)], idx_v)

    iota = lax.iota(jnp.int32, 16)
    tokbase = [[bt * 128 + bsg * 16 + iota for bsg in range(8)]
               for bt in range(2)]

    def g_copy(c, p):
      h, half = c // 2, c % 2
      return pltpu.make_async_copy(
          table_hbm.at[idx_v.at[h, pl.ds(half * _CT, _CT)]], rows_v.at[p],
          gsem[p])

    def s_copies(c, p):
      h, half = c // 2, c % 2
      off = bt0 + half * 2
      return [
          pltpu.make_async_copy(
              t_v.at[p, dt], out_hbm.at[h, dt, pl.ds(off, 2)], ssem[p])
          for dt in range(8)
      ]

    def transpose(gp, p):
      def tbody(j, _):
        dt, ds = j // 8, j % 8
        dv = lax.broadcast(j, (16,))
        for bt in range(2):
          for bsg in range(8):
            v = plsc.load_gather(rows_v.at[gp], [tokbase[bt][bsg], dv])
            t_v[p, dt, bt, ds, pl.ds(bsg * 16, 16)] = v
        return 0
      lax.fori_loop(0, 64, tbody, 0)

    def step(c, q, launch_gather, wait_scatter):
      g_copy(c, q).wait()
      if wait_scatter:
        for d in s_copies(0, q):
          d.wait()
      transpose(q, q)
      for d in s_copies(c, q):
        d.start()
      if launch_gather:
        g_copy(c + _RING, q).start()

    for q in range(_RING):
      g_copy(q, q).start()

    for q in range(_RING):
      step(q, q, True, False)

    def body(s, _):
      for q in range(_RING):
        step(s * _RING + q, q, True, True)
      return 0

    lax.fori_loop(1, _NCHUNK // _RING - 1, body, 0)

    for q in range(_RING):
      step(_NCHUNK - _RING + q, q, False, True)

    for q in range(_RING):
      for d in s_copies(0, q):
        d.wait()

  return k


_sc_gather = _make_sc_gather()


def kernel(tokens, embedding_weight):
  idxt = tokens.T.astype(jnp.int32) * 2
  table = jlayout.with_layout_constraint(
      embedding_weight, jlayout.Layout((0, 1), tiling=((8, 128),)))
  out5 = _sc_gather(table, idxt)
  return out5.transpose(2, 4, 0, 1, 3).reshape(BATCH, HIST_LEN, EMBED_DIM)


# speedup vs baseline: 1.5577x; 1.5577x over previous
import functools

import jax
import jax.numpy as jnp
from jax import lax
from jax.experimental import pallas as pl
from jax.experimental import layout as jlayout
from jax.experimental.pallas import tpu as pltpu
from jax.experimental.pallas import tpu_sc as plsc

VOCAB_SIZE = 1000000
EMBED_DIM = 64
BATCH = 16384
HIST_LEN = 50

_INFO = plsc.get_sparse_core_info()
_NC, _NS = _INFO.num_cores, _INFO.num_subcores
_NW = _NC * _NS
_B = BATCH * HIST_LEN
_BPW = _B // _NW
_RPW = BATCH // _NW
_CROWS = 4
_CH = _CROWS * HIST_LEN
_NCHUNK = _BPW // _CH
_RING = 4
_LOOK = 2
_NSUP = _NCHUNK // _RING


def _make_sc_gather():
  mesh = plsc.VectorSubcoreMesh(core_axis_name="c", subcore_axis_name="s")

  @functools.partial(
      pl.kernel,
      mesh=mesh,
      compiler_params=pltpu.CompilerParams(use_tc_tiling_on_sc=False),
      out_type=jax.ShapeDtypeStruct((BATCH, HIST_LEN, EMBED_DIM), jnp.float32),
      scratch_types=[
          pltpu.VMEM((_BPW,), jnp.int32),
          pltpu.VMEM((_RING, _CH, EMBED_DIM), jnp.float32),
          [pltpu.SemaphoreType.DMA] * _RING,
          [pltpu.SemaphoreType.DMA] * _RING,
      ],
  )
  def k(table_hbm, idx_hbm, out_hbm, idx_v, rows_v, gsem, ssem):
    wid = lax.axis_index("s") * _NC + lax.axis_index("c")
    base = wid * _BPW
    row0 = wid * _RPW
    pltpu.sync_copy(idx_hbm.at[pl.ds(base, _BPW)], idx_v)

    def g_copy(c, b):
      return pltpu.make_async_copy(
          table_hbm.at[idx_v.at[pl.ds(c * _CH, _CH)]], rows_v.at[b], gsem[b])

    def s_copies(c, b):
      return [
          pltpu.make_async_copy(
              rows_v.at[b].at[pl.ds(r * HIST_LEN, HIST_LEN)],
              out_hbm.at[row0 + c * _CROWS + r], ssem[b])
          for r in range(_CROWS)
      ]

    def step(c, b, launch):
      g_copy(c, b).wait()
      for d in s_copies(c, b):
        d.start()
      if launch:
        nb = (b + _LOOK) % _RING
        if launch == 2:
          for d in s_copies(0, nb):
            d.wait()
        g_copy(c + _LOOK, nb).start()

    for b in range(_LOOK):
      g_copy(b, b).start()

    for b in range(_RING):
      step(b, b, launch=1 if b + _LOOK < _RING else 2)

    def body(s, _):
      for b in range(_RING):
        step(s * _RING + b, b, launch=2)
      return 0

    lax.fori_loop(1, _NSUP - 1, body, 0)

    c0 = (_NSUP - 1) * _RING
    for b in range(_RING):
      step(c0 + b, b, launch=2 if c0 + b + _LOOK < _NCHUNK else 0)

    for b in range(_RING):
      for d in s_copies(0, b):
        d.wait()

  return k


_sc_gather = _make_sc_gather()


def kernel(tokens, embedding_weight):
  idx = tokens.reshape(_B).astype(jnp.int32) * 2
  table = jlayout.with_layout_constraint(
      embedding_weight, jlayout.Layout((0, 1), tiling=((8, 128),)))
  return _sc_gather(table, idx)
